# baseline (speedup 1.0000x reference)
import jax
import jax.numpy as jnp
from jax import lax
from jax.experimental import pallas as pl
from jax.experimental.pallas import tpu as pltpu
from jax.experimental.pallas import tpu_sc as plsc

BATCH = 16384
NUM_NODES = 1000000
NSUB = 16
RANGE = NUM_NODES // NSUB
RANGE_PAD = 62512
LIST_PAD = BATCH + 16
NU_LEN = BATCH + 16
MERGE_BLK = 131072


def _lane_iota():
    return lax.iota(jnp.int32, 16)


def _s0(v):
    return jnp.squeeze(lax.slice(v, (0,), (1,)))


def _sc_body(dst_hbm, times_hbm, nu_hbm,
             dst_v, times_v, tloc, mli, mtv, cnt_v, allcnt_v,
             zero_v, sem_in, sem_out, counts_sh):
    compact = mtv
    sid = lax.axis_index("s")
    base = sid * RANGE
    lanes = _lane_iota()

    cp_d = pltpu.make_async_copy(dst_hbm, dst_v, sem_in)
    cp_t = pltpu.make_async_copy(times_hbm, times_v, sem_in)
    cp_d.start()
    cp_t.start()

    neg1 = jnp.full((16,), -1.0, jnp.float32)

    def init_body(i, _):
        tloc[pl.ds(i * 16, 16)] = neg1
        return 0

    lax.fori_loop(0, RANGE_PAD // 16, init_body, 0, unroll=8)

    cp_d.wait()
    cp_t.wait()

    def filt(i, off):
        d = dst_v[pl.ds(i * 16, 16)]
        li = d - base
        mine = (li >= 0) & (li < RANGE)
        nmine = _s0(plsc.all_reduce_population_count(mine))

        @pl.when(nmine > 0)
        def _():
            t = times_v[pl.ds(i * 16, 16)]
            plsc.store_compressed(mli.at[pl.ds(off, 16)], li, mask=mine)
            plsc.store_compressed(mtv.at[pl.ds(off, 16)], t, mask=mine)

        return off + nmine

    mcount = lax.fori_loop(0, BATCH // 16, filt, jnp.int32(0), unroll=8)

    nv = (mcount + 15) // 16

    def rmw_outer(i, _):
        li = mli[pl.ds(i * 16, 16)]
        t = mtv[pl.ds(i * 16, 16)]
        valid = (i * 16 + lanes) < mcount
        lic = jnp.clip(li, 0, RANGE - 1)
        cnt, _ = plsc.scan_count(li, valid)
        cmin = jnp.min(jnp.where(valid, cnt, 1 << 30))
        cmax = jnp.max(jnp.where(valid, cnt, -(1 << 30)))
        npass = jnp.maximum(cmax - cmin + 1, 0)

        def rmw(c, _):
            sub = valid & (cnt == cmin + c)
            cur = plsc.load_gather(tloc, [lic], mask=sub)
            want = sub & (t > cur)
            plsc.store_scatter(tloc, [lic], t, mask=want)
            return 0

        lax.fori_loop(0, npass, rmw, 0)
        return 0

    lax.fori_loop(0, nv, rmw_outer, 0)

    def comp(i, off):
        v = tloc[pl.ds(i * 16, 16)]
        m = v >= 0.0
        plsc.store_compressed(compact.at[pl.ds(off, 16)], v, mask=m)
        return off + _s0(plsc.all_reduce_population_count(m))

    kw = lax.fori_loop(0, RANGE_PAD // 16, comp, jnp.int32(0), unroll=8)

    cnt_v[:] = jnp.full((16,), 0, jnp.int32) + kw
    pltpu.sync_copy(cnt_v, counts_sh.at[sid])
    plsc.subcore_barrier()
    pltpu.sync_copy(counts_sh, allcnt_v)

    kvec = plsc.load_gather(allcnt_v, [lanes, lanes * 0])
    off_w = jnp.sum(jnp.where(lanes < sid, kvec, 0))
    k_tot = jnp.sum(kvec)

    nchunks = (kw + 15) // 16

    def val_copy(c):
        idx = off_w + c * 16 + lanes
        valid = (c * 16 + lanes) < kw
        idx = jnp.where(valid, idx, BATCH + lanes)
        return pltpu.make_async_copy(compact.at[pl.ds(c * 16, 16)],
                                     nu_hbm.at[idx], sem_out)

    def fire_vals(c, _):
        val_copy(c).start()
        return 0

    lax.fori_loop(0, nchunks, fire_vals, 0)

    def drain_vals(c, _):
        val_copy(c).wait()
        return 0

    lax.fori_loop(0, nchunks, drain_vals, 0)

    zero_v[:] = jnp.zeros((16,), jnp.float32)
    tail_len = NU_LEN - k_tot
    share = (tail_len + NSUB - 1) // NSUB
    t0 = k_tot + sid * share
    t1 = jnp.minimum(t0 + share, NU_LEN)
    ztchunks = jnp.maximum((t1 - t0 + 15) // 16, 0)

    def tail_copy(c):
        idx = t0 + c * 16 + lanes
        valid = idx < t1
        idx = jnp.where(valid, idx, BATCH + lanes)
        return pltpu.make_async_copy(zero_v, nu_hbm.at[idx], sem_out)

    def fire_tail(c, _):
        tail_copy(c).start()
        return 0

    lax.fori_loop(0, ztchunks, fire_tail, 0)

    def drain_tail(c, _):
        tail_copy(c).wait()
        return 0

    lax.fori_loop(0, ztchunks, drain_tail, 0)


def _sc_dedup_segmax(dst_ids, times):
    mesh = plsc.VectorSubcoreMesh(core_axis_name="c", subcore_axis_name="s",
                                  num_cores=1)
    f = pl.kernel(
        _sc_body,
        mesh=mesh,
        compiler_params=pltpu.CompilerParams(needs_layout_passes=False,
                                             use_tc_tiling_on_sc=False),
        out_type=jax.ShapeDtypeStruct((NU_LEN,), jnp.float32),
        scratch_types=[
            pltpu.VMEM((BATCH,), jnp.int32),
            pltpu.VMEM((BATCH,), jnp.float32),
            pltpu.VMEM((RANGE_PAD,), jnp.float32),
            pltpu.VMEM((LIST_PAD,), jnp.int32),
            pltpu.VMEM((LIST_PAD,), jnp.float32),
            pltpu.VMEM((16,), jnp.int32),
            pltpu.VMEM((16, 16), jnp.int32),
            pltpu.VMEM((16,), jnp.float32),
            pltpu.SemaphoreType.DMA,
            pltpu.SemaphoreType.DMA,
            pltpu.VMEM_SHARED((16, 16), jnp.int32),
        ],
    )
    return f(dst_ids, times)


def _merge_body(last_ref, nu_ref, out_ref):
    out_ref[:] = jnp.maximum(last_ref[:], 0.0)

    @pl.when(pl.program_id(0) == 0)
    def _():
        out_ref[0:BATCH] = jnp.maximum(out_ref[0:BATCH], nu_ref[:])


def _merge(last_update, new_up):
    n = last_update.shape[0]
    return pl.pallas_call(
        _merge_body,
        grid=(pl.cdiv(n, MERGE_BLK),),
        in_specs=[
            pl.BlockSpec((MERGE_BLK,), lambda i: (i,)),
            pl.BlockSpec((BATCH,), lambda i: (0,)),
        ],
        out_specs=pl.BlockSpec((MERGE_BLK,), lambda i: (i,)),
        out_shape=jax.ShapeDtypeStruct((n,), jnp.float32),
    )(last_update, new_up)


def kernel(last_update, dst_ids, times):
    nu = _sc_dedup_segmax(dst_ids, times)
    return _merge(last_update, nu[:BATCH])


# speedup vs baseline: 1.0628x; 1.0628x over previous
import jax
import jax.numpy as jnp
from jax import lax
from jax.experimental import pallas as pl
from jax.experimental.pallas import tpu as pltpu
from jax.experimental.pallas import tpu_sc as plsc

BATCH = 16384
NUM_NODES = 1000000
NSUB = 16
RANGE = NUM_NODES // NSUB
RANGE_PAD = 62512
LIST_PAD = BATCH + 16
NU_LEN = BATCH + 16
MERGE_BLK = 131072


def _lane_iota():
    return lax.iota(jnp.int32, 16)


def _s0(v):
    return jnp.squeeze(lax.slice(v, (0,), (1,)))


def _sc_body(dst_hbm, times_hbm, nu_hbm,
             dst_v, times_v, tloc, mli, mtv, cnt_v, allcnt_v,
             zero_v, sem_in, sem_out, counts_sh):
    compact = mtv
    sid = lax.axis_index("s")
    base = sid * RANGE
    lanes = _lane_iota()

    cp_d = pltpu.make_async_copy(dst_hbm, dst_v, sem_in)
    cp_t = pltpu.make_async_copy(times_hbm, times_v, sem_in)
    cp_d.start()
    cp_t.start()

    neg1 = jnp.full((16,), -1.0, jnp.float32)

    def init_body(i, _):
        tloc[pl.ds(i * 16, 16)] = neg1
        return 0

    lax.fori_loop(0, RANGE_PAD // 16, init_body, 0, unroll=8)

    cp_d.wait()
    cp_t.wait()

    def filt(i, off):
        d = dst_v[pl.ds(i * 16, 16)]
        t = times_v[pl.ds(i * 16, 16)]
        li = d - base
        mine = (li >= 0) & (li < RANGE)
        plsc.store_compressed(mli.at[pl.ds(off, 16)], li, mask=mine)
        plsc.store_compressed(mtv.at[pl.ds(off, 16)], t, mask=mine)
        return off + _s0(plsc.all_reduce_population_count(mine))

    mcount = lax.fori_loop(0, BATCH // 16, filt, jnp.int32(0), unroll=8)

    nv = (mcount + 15) // 16

    def rmw_outer(i, _):
        li = mli[pl.ds(i * 16, 16)]
        t = mtv[pl.ds(i * 16, 16)]
        valid = (i * 16 + lanes) < mcount
        lic = jnp.clip(li, 0, RANGE - 1)
        cnt, _ = plsc.scan_count(li, valid)
        cmin = jnp.min(jnp.where(valid, cnt, 1 << 30))
        cmax = jnp.max(jnp.where(valid, cnt, -(1 << 30)))
        npass = jnp.maximum(cmax - cmin + 1, 0)

        def rmw(c, _):
            sub = valid & (cnt == cmin + c)
            cur = plsc.load_gather(tloc, [lic], mask=sub)
            want = sub & (t > cur)
            plsc.store_scatter(tloc, [lic], t, mask=want)
            return 0

        lax.fori_loop(0, npass, rmw, 0)
        return 0

    lax.fori_loop(0, nv, rmw_outer, 0)

    def comp(i, off):
        v = tloc[pl.ds(i * 16, 16)]
        m = v >= 0.0
        plsc.store_compressed(compact.at[pl.ds(off, 16)], v, mask=m)
        return off + _s0(plsc.all_reduce_population_count(m))

    kw = lax.fori_loop(0, RANGE_PAD // 16, comp, jnp.int32(0), unroll=8)

    cnt_v[:] = jnp.full((16,), 0, jnp.int32) + kw
    pltpu.sync_copy(cnt_v, counts_sh.at[sid])
    plsc.subcore_barrier()
    pltpu.sync_copy(counts_sh, allcnt_v)

    kvec = plsc.load_gather(allcnt_v, [lanes, lanes * 0])
    off_w = jnp.sum(jnp.where(lanes < sid, kvec, 0))
    k_tot = jnp.sum(kvec)

    nchunks = (kw + 15) // 16

    def val_copy(c):
        idx = off_w + c * 16 + lanes
        valid = (c * 16 + lanes) < kw
        idx = jnp.where(valid, idx, BATCH + lanes)
        return pltpu.make_async_copy(compact.at[pl.ds(c * 16, 16)],
                                     nu_hbm.at[idx], sem_out)

    def fire_vals(c, _):
        val_copy(c).start()
        return 0

    lax.fori_loop(0, nchunks, fire_vals, 0)

    def drain_vals(c, _):
        val_copy(c).wait()
        return 0

    lax.fori_loop(0, nchunks, drain_vals, 0)

    zero_v[:] = jnp.zeros((16,), jnp.float32)
    tail_len = NU_LEN - k_tot
    share = (tail_len + NSUB - 1) // NSUB
    t0 = k_tot + sid * share
    t1 = jnp.minimum(t0 + share, NU_LEN)
    ztchunks = jnp.maximum((t1 - t0 + 15) // 16, 0)

    def tail_copy(c):
        idx = t0 + c * 16 + lanes
        valid = idx < t1
        idx = jnp.where(valid, idx, BATCH + lanes)
        return pltpu.make_async_copy(zero_v, nu_hbm.at[idx], sem_out)

    def fire_tail(c, _):
        tail_copy(c).start()
        return 0

    lax.fori_loop(0, ztchunks, fire_tail, 0)

    def drain_tail(c, _):
        tail_copy(c).wait()
        return 0

    lax.fori_loop(0, ztchunks, drain_tail, 0)


def _sc_dedup_segmax(dst_ids, times):
    mesh = plsc.VectorSubcoreMesh(core_axis_name="c", subcore_axis_name="s",
                                  num_cores=1)
    f = pl.kernel(
        _sc_body,
        mesh=mesh,
        compiler_params=pltpu.CompilerParams(needs_layout_passes=False,
                                             use_tc_tiling_on_sc=False),
        out_type=jax.ShapeDtypeStruct((NU_LEN,), jnp.float32),
        scratch_types=[
            pltpu.VMEM((BATCH,), jnp.int32),
            pltpu.VMEM((BATCH,), jnp.float32),
            pltpu.VMEM((RANGE_PAD,), jnp.float32),
            pltpu.VMEM((LIST_PAD,), jnp.int32),
            pltpu.VMEM((LIST_PAD,), jnp.float32),
            pltpu.VMEM((16,), jnp.int32),
            pltpu.VMEM((16, 16), jnp.int32),
            pltpu.VMEM((16,), jnp.float32),
            pltpu.SemaphoreType.DMA,
            pltpu.SemaphoreType.DMA,
            pltpu.VMEM_SHARED((16, 16), jnp.int32),
        ],
    )
    return f(dst_ids, times)


def _merge_body(last_ref, nu_ref, out_ref):
    out_ref[:] = jnp.maximum(last_ref[:], 0.0)

    @pl.when(pl.program_id(0) == 0)
    def _():
        out_ref[0:BATCH] = jnp.maximum(out_ref[0:BATCH], nu_ref[:])


def _merge(last_update, new_up):
    n = last_update.shape[0]
    return pl.pallas_call(
        _merge_body,
        grid=(pl.cdiv(n, MERGE_BLK),),
        in_specs=[
            pl.BlockSpec((MERGE_BLK,), lambda i: (i,)),
            pl.BlockSpec((BATCH,), lambda i: (0,)),
        ],
        out_specs=pl.BlockSpec((MERGE_BLK,), lambda i: (i,)),
        out_shape=jax.ShapeDtypeStruct((n,), jnp.float32),
    )(last_update, new_up)


def kernel(last_update, dst_ids, times):
    nu = _sc_dedup_segmax(dst_ids, times)
    return _merge(last_update, nu[:BATCH])


# speedup vs baseline: 1.9713x; 1.8549x over previous
import jax
import jax.numpy as jnp
from jax import lax
from jax.experimental import pallas as pl
from jax.experimental.pallas import tpu as pltpu
from jax.experimental.pallas import tpu_sc as plsc

BATCH = 16384
NUM_NODES = 1000000
NSUB = 16
RANGE = NUM_NODES // NSUB
RANGE_PAD = 62512
LIST_PAD = BATCH + 16
NU_LEN = BATCH + 256
MERGE_BLK = 131072


def _lane_iota():
    return lax.iota(jnp.int32, 16)


def _s0(v):
    return jnp.squeeze(lax.slice(v, (0,), (1,)))


def _sc_body(dst_hbm, times_hbm, nu_hbm,
             dst_v, times_v, tloc, mli, mtv, cnt_v, allcnt_v,
             zero_v, sem_in, sem_out, counts_sh, nu_sh):
    compact = mtv
    sid = lax.axis_index("s")
    base = sid * RANGE
    lanes = _lane_iota()

    cp_d = pltpu.make_async_copy(dst_hbm, dst_v, sem_in)
    cp_t = pltpu.make_async_copy(times_hbm, times_v, sem_in)
    cp_d.start()
    cp_t.start()

    neg1 = jnp.full((16,), -1.0, jnp.float32)

    def init_body(i, _):
        tloc[pl.ds(i * 16, 16)] = neg1
        return 0

    lax.fori_loop(0, RANGE_PAD // 16, init_body, 0, unroll=8)

    cp_d.wait()
    cp_t.wait()

    def filt(i, off):
        d = dst_v[pl.ds(i * 16, 16)]
        t = times_v[pl.ds(i * 16, 16)]
        li = d - base
        mine = (li >= 0) & (li < RANGE)
        plsc.store_compressed(mli.at[pl.ds(off, 16)], li, mask=mine)
        plsc.store_compressed(mtv.at[pl.ds(off, 16)], t, mask=mine)
        return off + _s0(plsc.all_reduce_population_count(mine))

    mcount = lax.fori_loop(0, BATCH // 16, filt, jnp.int32(0), unroll=8)

    nv = (mcount + 15) // 16

    def rmw_outer(i, _):
        li = mli[pl.ds(i * 16, 16)]
        t = mtv[pl.ds(i * 16, 16)]
        valid = (i * 16 + lanes) < mcount
        lic = jnp.clip(li, 0, RANGE - 1)
        cnt, _ = plsc.scan_count(li, valid)
        cmin = jnp.min(jnp.where(valid, cnt, 1 << 30))
        cmax = jnp.max(jnp.where(valid, cnt, -(1 << 30)))
        npass = jnp.maximum(cmax - cmin + 1, 0)

        def rmw(c, _):
            sub = valid & (cnt == cmin + c)
            cur = plsc.load_gather(tloc, [lic], mask=sub)
            want = sub & (t > cur)
            plsc.store_scatter(tloc, [lic], t, mask=want)
            return 0

        lax.fori_loop(0, npass, rmw, 0)
        return 0

    lax.fori_loop(0, nv, rmw_outer, 0)

    def comp(i, off):
        v = tloc[pl.ds(i * 16, 16)]
        m = v >= 0.0
        plsc.store_compressed(compact.at[pl.ds(off, 16)], v, mask=m)
        return off + _s0(plsc.all_reduce_population_count(m))

    kw = lax.fori_loop(0, RANGE_PAD // 16, comp, jnp.int32(0), unroll=8)

    cnt_v[:] = jnp.full((16,), 0, jnp.int32) + kw
    pltpu.sync_copy(cnt_v, counts_sh.at[sid])
    plsc.subcore_barrier()
    pltpu.sync_copy(counts_sh, allcnt_v)

    kvec = plsc.load_gather(allcnt_v, [lanes, lanes * 0])
    off_w = jnp.sum(jnp.where(lanes < sid, kvec, 0))
    k_tot = jnp.sum(kvec)

    nchunks = (kw + 15) // 16

    def val_copy(c):
        idx = off_w + c * 16 + lanes
        valid = (c * 16 + lanes) < kw
        idx = jnp.where(valid, idx, BATCH + lanes)
        return pltpu.make_async_copy(compact.at[pl.ds(c * 16, 16)],
                                     nu_sh.at[idx], sem_out)

    def fire_vals(c, _):
        val_copy(c).start()
        return 0

    lax.fori_loop(0, nchunks, fire_vals, 0)

    def drain_vals(c, _):
        val_copy(c).wait()
        return 0

    lax.fori_loop(0, nchunks, drain_vals, 0)

    zero_v[:] = jnp.zeros((16,), jnp.float32)
    tail_len = NU_LEN - k_tot
    share = (tail_len + NSUB - 1) // NSUB
    t0 = k_tot + sid * share
    t1 = jnp.minimum(t0 + share, NU_LEN)
    ztchunks = jnp.maximum((t1 - t0 + 15) // 16, 0)

    def tail_copy(c):
        idx = t0 + c * 16 + lanes
        valid = idx < t1
        idx = jnp.where(valid, idx, BATCH + lanes)
        return pltpu.make_async_copy(zero_v, nu_sh.at[idx], sem_out)

    def fire_tail(c, _):
        tail_copy(c).start()
        return 0

    lax.fori_loop(0, ztchunks, fire_tail, 0)

    def drain_tail(c, _):
        tail_copy(c).wait()
        return 0

    lax.fori_loop(0, ztchunks, drain_tail, 0)

    plsc.subcore_barrier()
    sl = NU_LEN // NSUB
    pltpu.sync_copy(nu_sh.at[pl.ds(sid * sl, sl)],
                    nu_hbm.at[pl.ds(sid * sl, sl)])


def _sc_dedup_segmax(dst_ids, times):
    mesh = plsc.VectorSubcoreMesh(core_axis_name="c", subcore_axis_name="s",
                                  num_cores=1)
    f = pl.kernel(
        _sc_body,
        mesh=mesh,
        compiler_params=pltpu.CompilerParams(needs_layout_passes=False,
                                             use_tc_tiling_on_sc=False),
        out_type=jax.ShapeDtypeStruct((NU_LEN,), jnp.float32),
        scratch_types=[
            pltpu.VMEM((BATCH,), jnp.int32),
            pltpu.VMEM((BATCH,), jnp.float32),
            pltpu.VMEM((RANGE_PAD,), jnp.float32),
            pltpu.VMEM((LIST_PAD,), jnp.int32),
            pltpu.VMEM((LIST_PAD,), jnp.float32),
            pltpu.VMEM((16,), jnp.int32),
            pltpu.VMEM((16, 16), jnp.int32),
            pltpu.VMEM((16,), jnp.float32),
            pltpu.SemaphoreType.DMA,
            pltpu.SemaphoreType.DMA,
            pltpu.VMEM_SHARED((16, 16), jnp.int32),
            pltpu.VMEM_SHARED((NU_LEN,), jnp.float32),
        ],
    )
    return f(dst_ids, times)


def _merge_body(last_ref, nu_ref, out_ref):
    out_ref[:] = jnp.maximum(last_ref[:], 0.0)

    @pl.when(pl.program_id(0) == 0)
    def _():
        out_ref[0:BATCH] = jnp.maximum(out_ref[0:BATCH], nu_ref[:])


def _merge(last_update, new_up):
    n = last_update.shape[0]
    return pl.pallas_call(
        _merge_body,
        grid=(pl.cdiv(n, MERGE_BLK),),
        in_specs=[
            pl.BlockSpec((MERGE_BLK,), lambda i: (i,)),
            pl.BlockSpec((BATCH,), lambda i: (0,)),
        ],
        out_specs=pl.BlockSpec((MERGE_BLK,), lambda i: (i,)),
        out_shape=jax.ShapeDtypeStruct((n,), jnp.float32),
    )(last_update, new_up)


def kernel(last_update, dst_ids, times):
    nu = _sc_dedup_segmax(dst_ids, times)
    return _merge(last_update, nu[:BATCH])


# speedup vs baseline: 2.0202x; 1.0248x over previous
import jax
import jax.numpy as jnp
from jax import lax
from jax.experimental import pallas as pl
from jax.experimental.pallas import tpu as pltpu
from jax.experimental.pallas import tpu_sc as plsc

BATCH = 16384
NUM_NODES = 1000000
NSUB = 16
RANGE = NUM_NODES // NSUB
RANGE_PAD = 62512
LIST_PAD = BATCH + 16
NU_LEN = BATCH + 256
MERGE_BLK = 131072


def _lane_iota():
    return lax.iota(jnp.int32, 16)


def _s0(v):
    return jnp.squeeze(lax.slice(v, (0,), (1,)))


def _sc_body(dst_hbm, times_hbm, nu_hbm,
             dst_v, times_v, tloc, mli, mtv, cnt_v, allcnt_v,
             zero_v, sem_in, sem_out, counts_sh, nu_sh):
    compact = mtv
    sid = lax.axis_index("s")
    base = sid * RANGE
    lanes = _lane_iota()

    cp_d = pltpu.make_async_copy(dst_hbm, dst_v, sem_in)
    cp_t = pltpu.make_async_copy(times_hbm, times_v, sem_in)
    cp_d.start()
    cp_t.start()

    neg1 = jnp.full((16,), -1.0, jnp.float32)

    def init_body(i, _):
        tloc[pl.ds(i * 16, 16)] = neg1
        return 0

    lax.fori_loop(0, RANGE_PAD // 16, init_body, 0, unroll=8)

    cp_d.wait()
    cp_t.wait()

    def filt(i, off):
        d = dst_v[pl.ds(i * 16, 16)]
        t = times_v[pl.ds(i * 16, 16)]
        li = d - base
        mine = (li >= 0) & (li < RANGE)
        plsc.store_compressed(mli.at[pl.ds(off, 16)], li, mask=mine)
        plsc.store_compressed(mtv.at[pl.ds(off, 16)], t, mask=mine)
        return off + _s0(plsc.all_reduce_population_count(mine))

    mcount = lax.fori_loop(0, BATCH // 16, filt, jnp.int32(0), unroll=8)

    nv = (mcount + 15) // 16

    def rmw_outer(i, _):
        li = mli[pl.ds(i * 16, 16)]
        t = mtv[pl.ds(i * 16, 16)]
        valid = (i * 16 + lanes) < mcount
        lic = jnp.clip(li, 0, RANGE - 1)
        cnt, _ = plsc.scan_count(li, valid)
        cmin = jnp.min(jnp.where(valid, cnt, 1 << 30))
        cmax = jnp.max(jnp.where(valid, cnt, -(1 << 30)))
        npass = jnp.maximum(cmax - cmin + 1, 0)

        def rmw(c, _):
            sub = valid & (cnt == cmin + c)
            cur = plsc.load_gather(tloc, [lic], mask=sub)
            want = sub & (t > cur)
            plsc.store_scatter(tloc, [lic], t, mask=want)
            return 0

        lax.fori_loop(0, npass, rmw, 0)
        return 0

    lax.fori_loop(0, nv, rmw_outer, 0)

    def comp(i, off):
        v = tloc[pl.ds(i * 16, 16)]
        m = v >= 0.0
        plsc.store_compressed(compact.at[pl.ds(off, 16)], v, mask=m)
        return off + _s0(plsc.all_reduce_population_count(m))

    kw = lax.fori_loop(0, RANGE_PAD // 16, comp, jnp.int32(0), unroll=8)

    cnt_v[:] = jnp.full((16,), 0, jnp.int32) + kw
    pltpu.sync_copy(cnt_v, counts_sh.at[sid])
    plsc.subcore_barrier()
    pltpu.sync_copy(counts_sh, allcnt_v)

    kvec = plsc.load_gather(allcnt_v, [lanes, lanes * 0])
    off_w = jnp.sum(jnp.where(lanes < sid, kvec, 0))
    k_tot = jnp.sum(kvec)

    nchunks = (kw + 15) // 16

    def val_copy(c):
        idx = off_w + c * 16 + lanes
        valid = (c * 16 + lanes) < kw
        idx = jnp.where(valid, idx, BATCH + lanes)
        return pltpu.make_async_copy(compact.at[pl.ds(c * 16, 16)],
                                     nu_sh.at[idx], sem_out)

    def fire_vals(c, _):
        val_copy(c).start()
        return 0

    lax.fori_loop(0, nchunks, fire_vals, 0)

    def drain_vals(c, _):
        val_copy(c).wait()
        return 0

    lax.fori_loop(0, nchunks, drain_vals, 0)

    zero_v[:] = jnp.zeros((16,), jnp.float32)
    tail_len = NU_LEN - k_tot
    share = (tail_len + NSUB - 1) // NSUB
    t0 = k_tot + sid * share
    t1 = jnp.minimum(t0 + share, NU_LEN)
    ztchunks = jnp.maximum((t1 - t0 + 15) // 16, 0)

    def tail_copy(c):
        idx = t0 + c * 16 + lanes
        valid = idx < t1
        idx = jnp.where(valid, idx, BATCH + lanes)
        return pltpu.make_async_copy(zero_v, nu_sh.at[idx], sem_out)

    def fire_tail(c, _):
        tail_copy(c).start()
        return 0

    lax.fori_loop(0, ztchunks, fire_tail, 0)

    def drain_tail(c, _):
        tail_copy(c).wait()
        return 0

    lax.fori_loop(0, ztchunks, drain_tail, 0)

    plsc.subcore_barrier()
    sl = NU_LEN // NSUB
    pltpu.sync_copy(nu_sh.at[pl.ds(sid * sl, sl)],
                    nu_hbm.at[pl.ds(sid * sl, sl)])


def _sc_dedup_segmax(dst_ids, times):
    mesh = plsc.VectorSubcoreMesh(core_axis_name="c", subcore_axis_name="s",
                                  num_cores=1)
    f = pl.kernel(
        _sc_body,
        mesh=mesh,
        compiler_params=pltpu.CompilerParams(needs_layout_passes=False,
                                             use_tc_tiling_on_sc=False),
        out_type=jax.ShapeDtypeStruct((NU_LEN,), jnp.float32),
        scratch_types=[
            pltpu.VMEM((BATCH,), jnp.int32),
            pltpu.VMEM((BATCH,), jnp.float32),
            pltpu.VMEM((RANGE_PAD,), jnp.float32),
            pltpu.VMEM((LIST_PAD,), jnp.int32),
            pltpu.VMEM((LIST_PAD,), jnp.float32),
            pltpu.VMEM((16,), jnp.int32),
            pltpu.VMEM((16, 16), jnp.int32),
            pltpu.VMEM((16,), jnp.float32),
            pltpu.SemaphoreType.DMA,
            pltpu.SemaphoreType.DMA,
            pltpu.VMEM_SHARED((16, 16), jnp.int32),
            pltpu.VMEM_SHARED((NU_LEN,), jnp.float32),
        ],
    )
    return f(dst_ids, times)


def _merge_body(last_ref, nu_ref, out_ref):
    out_ref[:] = jnp.maximum(last_ref[:], 0.0)

    @pl.when(pl.program_id(0) == 0)
    def _():
        out_ref[0:BATCH] = jnp.maximum(out_ref[0:BATCH], nu_ref[:])


def _merge(last_update, new_up):
    n = last_update.shape[0]
    return pl.pallas_call(
        _merge_body,
        grid=(pl.cdiv(n, MERGE_BLK),),
        in_specs=[
            pl.BlockSpec((MERGE_BLK,), lambda i: (i,)),
            pl.BlockSpec((BATCH,), lambda i: (0,)),
        ],
        out_specs=pl.BlockSpec((MERGE_BLK,), lambda i: (i,)),
        out_shape=jax.ShapeDtypeStruct((n,), jnp.float32),
    )(last_update, new_up)


def kernel(last_update, dst_ids, times):
    nu = _sc_dedup_segmax(dst_ids, times)
    return _merge(last_update, nu)


# speedup vs baseline: 2.0452x; 1.0124x over previous
import jax
import jax.numpy as jnp
from jax import lax
from jax.experimental import pallas as pl
from jax.experimental.pallas import tpu as pltpu
from jax.experimental.pallas import tpu_sc as plsc

BATCH = 16384
NUM_NODES = 1000000
NSUB = 16
RANGE = NUM_NODES // NSUB
RANGE_PAD = 62512
LIST_PAD = BATCH + 16
NU_LEN = BATCH + 256
MERGE_BLK = 131072


def _lane_iota():
    return lax.iota(jnp.int32, 16)


def _s0(v):
    return jnp.squeeze(lax.slice(v, (0,), (1,)))


def _sc_body(dst_hbm, times_hbm, nu_hbm,
             dst_v, times_v, tloc, mli, mtv, cnt_v, allcnt_v,
             zero_v, sem_in, sem_out, counts_sh, nu_sh):
    compact = mtv
    sid = lax.axis_index("s")
    base = sid * RANGE
    lanes = _lane_iota()

    cp_d = pltpu.make_async_copy(dst_hbm, dst_v, sem_in)
    cp_t = pltpu.make_async_copy(times_hbm, times_v, sem_in)
    cp_d.start()
    cp_t.start()

    neg1 = jnp.full((16,), -1.0, jnp.float32)

    def init_body(i, _):
        tloc[pl.ds(i * 16, 16)] = neg1
        return 0

    lax.fori_loop(0, RANGE_PAD // 16, init_body, 0, unroll=8)

    cp_d.wait()
    cp_t.wait()

    def filt(i, off):
        d = dst_v[pl.ds(i * 16, 16)]
        t = times_v[pl.ds(i * 16, 16)]
        li = d - base
        mine = (li >= 0) & (li < RANGE)
        plsc.store_compressed(mli.at[pl.ds(off, 16)], li, mask=mine)
        plsc.store_compressed(mtv.at[pl.ds(off, 16)], t, mask=mine)
        return off + _s0(plsc.all_reduce_population_count(mine))

    mcount = lax.fori_loop(0, BATCH // 16, filt, jnp.int32(0), unroll=8)

    nv = (mcount + 15) // 16

    def rmw_outer(i, _):
        li = mli[pl.ds(i * 16, 16)]
        t = mtv[pl.ds(i * 16, 16)]
        valid = (i * 16 + lanes) < mcount
        lic = jnp.clip(li, 0, RANGE - 1)
        cnt, _ = plsc.scan_count(li, valid)
        cmin = jnp.min(jnp.where(valid, cnt, 1 << 30))
        cmax = jnp.max(jnp.where(valid, cnt, -(1 << 30)))
        npass = jnp.maximum(cmax - cmin + 1, 0)

        def rmw(c, _):
            sub = valid & (cnt == cmin + c)
            cur = plsc.load_gather(tloc, [lic], mask=sub)
            want = sub & (t > cur)
            plsc.store_scatter(tloc, [lic], t, mask=want)
            return 0

        lax.fori_loop(0, npass, rmw, 0)
        return 0

    lax.fori_loop(0, nv, rmw_outer, 0)

    def comp(i, off):
        v = tloc[pl.ds(i * 16, 16)]
        m = v >= 0.0
        plsc.store_compressed(compact.at[pl.ds(off, 16)], v, mask=m)
        return off + _s0(plsc.all_reduce_population_count(m))

    kw = lax.fori_loop(0, RANGE_PAD // 16, comp, jnp.int32(0), unroll=16)

    cnt_v[:] = jnp.full((16,), 0, jnp.int32) + kw
    pltpu.sync_copy(cnt_v, counts_sh.at[sid])
    plsc.subcore_barrier()
    pltpu.sync_copy(counts_sh, allcnt_v)

    kvec = plsc.load_gather(allcnt_v, [lanes, lanes * 0])
    off_w = jnp.sum(jnp.where(lanes < sid, kvec, 0))
    k_tot = jnp.sum(kvec)

    nchunks = (kw + 15) // 16

    def val_copy(c):
        idx = off_w + c * 16 + lanes
        valid = (c * 16 + lanes) < kw
        idx = jnp.where(valid, idx, BATCH + lanes)
        return pltpu.make_async_copy(compact.at[pl.ds(c * 16, 16)],
                                     nu_sh.at[idx], sem_out)

    def fire_vals(c, _):
        val_copy(c).start()
        return 0

    lax.fori_loop(0, nchunks, fire_vals, 0)

    def drain_vals(c, _):
        val_copy(c).wait()
        return 0

    lax.fori_loop(0, nchunks, drain_vals, 0)

    zero_v[:] = jnp.zeros((16,), jnp.float32)
    tail_len = NU_LEN - k_tot
    share = (tail_len + NSUB - 1) // NSUB
    t0 = k_tot + sid * share
    t1 = jnp.minimum(t0 + share, NU_LEN)
    ztchunks = jnp.maximum((t1 - t0 + 15) // 16, 0)

    def tail_copy(c):
        idx = t0 + c * 16 + lanes
        valid = idx < t1
        idx = jnp.where(valid, idx, BATCH + lanes)
        return pltpu.make_async_copy(zero_v, nu_sh.at[idx], sem_out)

    def fire_tail(c, _):
        tail_copy(c).start()
        return 0

    lax.fori_loop(0, ztchunks, fire_tail, 0)

    def drain_tail(c, _):
        tail_copy(c).wait()
        return 0

    lax.fori_loop(0, ztchunks, drain_tail, 0)

    plsc.subcore_barrier()
    sl = NU_LEN // NSUB
    pltpu.sync_copy(nu_sh.at[pl.ds(sid * sl, sl)],
                    nu_hbm.at[pl.ds(sid * sl, sl)])


def _sc_dedup_segmax(dst_ids, times):
    mesh = plsc.VectorSubcoreMesh(core_axis_name="c", subcore_axis_name="s",
                                  num_cores=1)
    f = pl.kernel(
        _sc_body,
        mesh=mesh,
        compiler_params=pltpu.CompilerParams(needs_layout_passes=False,
                                             use_tc_tiling_on_sc=False),
        out_type=jax.ShapeDtypeStruct((NU_LEN,), jnp.float32),
        scratch_types=[
            pltpu.VMEM((BATCH,), jnp.int32),
            pltpu.VMEM((BATCH,), jnp.float32),
            pltpu.VMEM((RANGE_PAD,), jnp.float32),
            pltpu.VMEM((LIST_PAD,), jnp.int32),
            pltpu.VMEM((LIST_PAD,), jnp.float32),
            pltpu.VMEM((16,), jnp.int32),
            pltpu.VMEM((16, 16), jnp.int32),
            pltpu.VMEM((16,), jnp.float32),
            pltpu.SemaphoreType.DMA,
            pltpu.SemaphoreType.DMA,
            pltpu.VMEM_SHARED((16, 16), jnp.int32),
            pltpu.VMEM_SHARED((NU_LEN,), jnp.float32),
        ],
    )
    return f(dst_ids, times)


def _merge_body(last_ref, nu_ref, out_ref):
    out_ref[:] = jnp.maximum(last_ref[:], 0.0)

    @pl.when(pl.program_id(0) == 0)
    def _():
        out_ref[0:BATCH] = jnp.maximum(out_ref[0:BATCH], nu_ref[:])


def _merge(last_update, new_up):
    n = last_update.shape[0]
    return pl.pallas_call(
        _merge_body,
        grid=(pl.cdiv(n, MERGE_BLK),),
        in_specs=[
            pl.BlockSpec((MERGE_BLK,), lambda i: (i,)),
            pl.BlockSpec((BATCH,), lambda i: (0,)),
        ],
        out_specs=pl.BlockSpec((MERGE_BLK,), lambda i: (i,)),
        out_shape=jax.ShapeDtypeStruct((n,), jnp.float32),
    )(last_update, new_up)


def kernel(last_update, dst_ids, times):
    nu = _sc_dedup_segmax(dst_ids, times)
    return _merge(last_update, nu)
